# Initial kernel scaffold; baseline (speedup 1.0000x reference)
import jax
import jax.numpy as jnp
from jax.experimental import pallas as pl


def kernel(x, relative_position_bias_table, relative_position_index):
    raise NotImplementedError("write your pallas kernel here")


# speedup vs baseline: 2.2978x; 2.2978x over previous
import functools

import jax
import jax.numpy as jnp
from jax import lax
from jax.experimental import pallas as pl
from jax.experimental.pallas import tpu as pltpu
from jax.experimental.pallas import tpu_sc as plsc

_H = 16
_TW = 256
_P = _TW * _TW


def _sc_gather_bias(tab_flat, idx_flat):
    info = plsc.get_sparse_core_info()
    nc, ns = info.num_cores, info.num_subcores
    nw = nc * ns
    halves = nw // _H
    per = _P // halves
    tab_len = tab_flat.shape[0]
    mesh = plsc.VectorSubcoreMesh(core_axis_name="c", subcore_axis_name="s")

    @functools.partial(
        pl.kernel,
        mesh=mesh,
        out_type=jax.ShapeDtypeStruct((_H, _P), jnp.float32),
        compiler_params=pltpu.CompilerParams(needs_layout_passes=False),
        scratch_types=[
            pltpu.VMEM((per,), jnp.int32),
            pltpu.VMEM((tab_len,), jnp.float32),
            pltpu.VMEM((per,), jnp.float32),
        ],
    )
    def k(tab_hbm, idx_hbm, out_hbm, idx_v, tab_v, out_v):
        wid = lax.axis_index("s") * nc + lax.axis_index("c")
        h = wid % _H
        base = (wid // _H) * per
        pltpu.sync_copy(idx_hbm.at[pl.ds(base, per)], idx_v)
        pltpu.sync_copy(tab_hbm, tab_v)

        def body(kk, carry):
            i16 = idx_v[pl.ds(kk * 16, 16)]
            out_v[pl.ds(kk * 16, 16)] = plsc.load_gather(tab_v, [i16 * _H + h])
            return carry

        lax.fori_loop(0, per // 16, body, 0, unroll=8)
        pltpu.sync_copy(out_v, out_hbm.at[h, pl.ds(base, per)])

    return k(tab_flat, idx_flat)


def _tc_add(x, bias):
    b_count = x.shape[0]

    def body(x_ref, b_ref, o_ref):
        o_ref[...] = x_ref[...] + b_ref[...]

    return pl.pallas_call(
        body,
        grid=(b_count,),
        in_specs=[
            pl.BlockSpec((1, _H, _TW, _TW), lambda b: (b, 0, 0, 0)),
            pl.BlockSpec((1, _H, _TW, _TW), lambda b: (0, 0, 0, 0)),
        ],
        out_specs=pl.BlockSpec((1, _H, _TW, _TW), lambda b: (b, 0, 0, 0)),
        out_shape=jax.ShapeDtypeStruct(x.shape, x.dtype),
    )(x, bias)


def kernel(x, relative_position_bias_table, relative_position_index):
    tab_flat = relative_position_bias_table.reshape(-1)
    idx_flat = relative_position_index.reshape(-1)
    bias = _sc_gather_bias(tab_flat, idx_flat)
    return _tc_add(x, bias.reshape(1, _H, _TW, _TW))


# speedup vs baseline: 2.6733x; 1.1634x over previous
import functools

import jax
import jax.numpy as jnp
from jax import lax
from jax.experimental import pallas as pl
from jax.experimental.pallas import tpu as pltpu
from jax.experimental.pallas import tpu_sc as plsc

_H = 16
_TW = 256
_P = _TW * _TW


def _sc_gather_bias(tab_t, idx_flat):
    info = plsc.get_sparse_core_info()
    nc, ns = info.num_cores, info.num_subcores
    nw = nc * ns
    halves = nw // _H
    per = _P // halves
    tab_row = tab_t.shape[1]
    mesh = plsc.VectorSubcoreMesh(core_axis_name="c", subcore_axis_name="s")

    @functools.partial(
        pl.kernel,
        mesh=mesh,
        out_type=jax.ShapeDtypeStruct((_H, _P), jnp.float32),
        compiler_params=pltpu.CompilerParams(needs_layout_passes=False),
        scratch_types=[
            pltpu.VMEM((per,), jnp.int32),
            pltpu.VMEM((tab_row,), jnp.float32),
            pltpu.VMEM((per,), jnp.float32),
        ],
    )
    def k(tab_hbm, idx_hbm, out_hbm, idx_v, tab_v, out_v):
        wid = lax.axis_index("s") * nc + lax.axis_index("c")
        h = wid % _H
        base = (wid // _H) * per
        pltpu.sync_copy(idx_hbm.at[pl.ds(base, per)], idx_v)
        pltpu.sync_copy(tab_hbm.at[h], tab_v)

        @plsc.parallel_loop(0, per, step=16, unroll=8)
        def body(i):
            out_v[pl.ds(i, 16)] = plsc.load_gather(tab_v, [idx_v[pl.ds(i, 16)]])

        pltpu.sync_copy(out_v, out_hbm.at[h, pl.ds(base, per)])

    return k(tab_t, idx_flat)


def _tc_add(x, bias, hb=8):
    b_count = x.shape[0]

    def body(x_ref, b_ref, o_ref):
        o_ref[...] = x_ref[...] + b_ref[...]

    return pl.pallas_call(
        body,
        grid=(_H // hb, b_count),
        in_specs=[
            pl.BlockSpec((1, hb, _TW, _TW), lambda h, b: (b, h, 0, 0)),
            pl.BlockSpec((1, hb, _TW, _TW), lambda h, b: (0, h, 0, 0)),
        ],
        out_specs=pl.BlockSpec((1, hb, _TW, _TW), lambda h, b: (b, h, 0, 0)),
        out_shape=jax.ShapeDtypeStruct(x.shape, x.dtype),
    )(x, bias)


def kernel(x, relative_position_bias_table, relative_position_index):
    tab_t = jnp.pad(relative_position_bias_table.T, ((0, 0), (0, 7)))
    idx_flat = relative_position_index.reshape(-1)
    bias = _sc_gather_bias(tab_t, idx_flat)
    return _tc_add(x, bias.reshape(1, _H, _TW, _TW))


# speedup vs baseline: 2.9152x; 1.0905x over previous
import functools

import jax
import jax.numpy as jnp
from jax import lax
from jax.experimental import pallas as pl
from jax.experimental.pallas import tpu as pltpu
from jax.experimental.pallas import tpu_sc as plsc

_H = 16
_TW = 256
_P = _TW * _TW


def _sc_gather_bias(tab_t, idx_flat):
    info = plsc.get_sparse_core_info()
    nc, ns = info.num_cores, info.num_subcores
    nw = nc * ns
    halves = nw // _H
    per = _P // halves
    tab_row = tab_t.shape[1]
    mesh = plsc.VectorSubcoreMesh(core_axis_name="c", subcore_axis_name="s")

    @functools.partial(
        pl.kernel,
        mesh=mesh,
        out_type=jax.ShapeDtypeStruct((_H, _P), jnp.float32),
        compiler_params=pltpu.CompilerParams(needs_layout_passes=False),
        scratch_types=[
            pltpu.VMEM((per,), jnp.int32),
            pltpu.VMEM((tab_row,), jnp.float32),
            pltpu.VMEM((per,), jnp.float32),
        ],
    )
    def k(tab_hbm, idx_hbm, out_hbm, idx_v, tab_v, out_v):
        wid = lax.axis_index("s") * nc + lax.axis_index("c")
        h = wid % _H
        base = (wid // _H) * per
        pltpu.sync_copy(idx_hbm.at[pl.ds(base, per)], idx_v)
        pltpu.sync_copy(tab_hbm.at[h], tab_v)

        @plsc.parallel_loop(0, per, step=16, unroll=8)
        def body(i):
            out_v[pl.ds(i, 16)] = plsc.load_gather(tab_v, [idx_v[pl.ds(i, 16)]])

        pltpu.sync_copy(out_v, out_hbm.at[h, pl.ds(base, per)])

    return k(tab_t, idx_flat)


def _tc_add(x, bias, bb=2):
    b_count = x.shape[0]

    def body(x_ref, b_ref, o_ref):
        o_ref[...] = x_ref[...] + b_ref[...]

    return pl.pallas_call(
        body,
        grid=(b_count // bb,),
        in_specs=[
            pl.BlockSpec((bb, _H, _TW, _TW), lambda b: (b, 0, 0, 0)),
            pl.BlockSpec((1, _H, _TW, _TW), lambda b: (0, 0, 0, 0)),
        ],
        out_specs=pl.BlockSpec((bb, _H, _TW, _TW), lambda b: (b, 0, 0, 0)),
        out_shape=jax.ShapeDtypeStruct(x.shape, x.dtype),
    )(x, bias)


def kernel(x, relative_position_bias_table, relative_position_index):
    tab_t = jnp.pad(relative_position_bias_table.T, ((0, 0), (0, 7)))
    idx_flat = relative_position_index.reshape(-1)
    bias = _sc_gather_bias(tab_t, idx_flat)
    return _tc_add(x, bias.reshape(1, _H, _TW, _TW))


# speedup vs baseline: 2.9310x; 1.0054x over previous
import functools

import jax
import jax.numpy as jnp
from jax import lax
from jax.experimental import pallas as pl
from jax.experimental.pallas import tpu as pltpu
from jax.experimental.pallas import tpu_sc as plsc

_H = 16
_TW = 256
_P = _TW * _TW


def _sc_gather_bias(tab_t, idx_flat):
    info = plsc.get_sparse_core_info()
    nc, ns = info.num_cores, info.num_subcores
    nw = nc * ns
    halves = nw // _H
    per = _P // halves
    tab_row = tab_t.shape[1]
    mesh = plsc.VectorSubcoreMesh(core_axis_name="c", subcore_axis_name="s")

    @functools.partial(
        pl.kernel,
        mesh=mesh,
        out_type=jax.ShapeDtypeStruct((_H, _P), jnp.float32),
        compiler_params=pltpu.CompilerParams(needs_layout_passes=False),
        scratch_types=[
            pltpu.VMEM((per,), jnp.int32),
            pltpu.VMEM((tab_row,), jnp.float32),
            pltpu.VMEM((per,), jnp.float32),
            pltpu.SemaphoreType.DMA,
            pltpu.SemaphoreType.DMA,
            pltpu.SemaphoreType.DMA,
            pltpu.SemaphoreType.DMA,
        ],
    )
    def k(tab_hbm, idx_hbm, out_hbm, idx_v, tab_v, out_v, s0, s1, st, so):
        wid = lax.axis_index("s") * nc + lax.axis_index("c")
        h = wid % _H
        base = (wid // _H) * per
        nch = 4
        ch = per // nch
        sems = (s0, s1)
        tab_cp = pltpu.make_async_copy(tab_hbm.at[h], tab_v, st)
        tab_cp.start()
        idx_cp = [
            pltpu.make_async_copy(
                idx_hbm.at[pl.ds(base + kk * ch, ch)],
                idx_v.at[pl.ds(kk * ch, ch)], sems[kk % 2])
            for kk in range(nch)
        ]
        out_cp = [
            pltpu.make_async_copy(
                out_v.at[pl.ds(kk * ch, ch)],
                out_hbm.at[h, pl.ds(base + kk * ch, ch)], so)
            for kk in range(nch)
        ]
        idx_cp[0].start()
        idx_cp[1].start()
        tab_cp.wait()
        for kk in range(nch):
            idx_cp[kk].wait()
            if kk + 2 < nch:
                idx_cp[kk + 2].start()
            lo = kk * ch

            @plsc.parallel_loop(lo, lo + ch, step=16, unroll=8)
            def body(i):
                out_v[pl.ds(i, 16)] = plsc.load_gather(
                    tab_v, [idx_v[pl.ds(i, 16)]])

            out_cp[kk].start()
        for kk in range(nch):
            out_cp[kk].wait()

    return k(tab_t, idx_flat)


def _tc_add(x, bias, bb=2):
    b_count = x.shape[0]

    def body(x_ref, b_ref, o_ref):
        o_ref[...] = x_ref[...] + b_ref[...]

    return pl.pallas_call(
        body,
        grid=(b_count // bb,),
        in_specs=[
            pl.BlockSpec((bb, _H, _TW, _TW), lambda b: (b, 0, 0, 0)),
            pl.BlockSpec((1, _H, _TW, _TW), lambda b: (0, 0, 0, 0)),
        ],
        out_specs=pl.BlockSpec((bb, _H, _TW, _TW), lambda b: (b, 0, 0, 0)),
        out_shape=jax.ShapeDtypeStruct(x.shape, x.dtype),
    )(x, bias)


def kernel(x, relative_position_bias_table, relative_position_index):
    tab_t = jnp.pad(relative_position_bias_table.T, ((0, 0), (0, 7)))
    idx_flat = relative_position_index.reshape(-1)
    bias = _sc_gather_bias(tab_t, idx_flat)
    return _tc_add(x, bias.reshape(1, _H, _TW, _TW))
